# Initial kernel scaffold; baseline (speedup 1.0000x reference)
import jax
import jax.numpy as jnp
from jax.experimental import pallas as pl


def kernel(x, edge_index, conv0_Wl, conv0_Wr, conv0_att, conv0_b, bn0_g, bn0_b, conv1_Wl, conv1_Wr, conv1_att, conv1_b, bn1_g, bn1_b, cls_W, cls_b):
    raise NotImplementedError("write your pallas kernel here")


# speedup vs baseline: 19.7451x; 19.7451x over previous
import jax
import jax.numpy as jnp
from jax import lax
from jax.experimental import pallas as pl
from jax.experimental.pallas import tpu as pltpu
from jax.experimental.pallas import tpu_sc as plsc

N = 10000
HID = 16
HEADS = 8
F = HEADS * HID
NEG = 0.2
EPS = 1e-5

NC = 2
NS = 16
NW = NC * NS
B = 128
NP = 10240
RPT = NP // NS

_mesh = plsc.VectorSubcoreMesh(core_axis_name="c", subcore_axis_name="s")
_SC_PARAMS = pltpu.CompilerParams(needs_layout_passes=False)


def _score_body(xl, xr, src, dstg, dsts, attf,
                ex, den,
                xlb, xrb, exb, srcb, dgb, dsb, idxb, attv, attb, zb, den_sh,
                sem0, sem1):
    c = lax.axis_index("c")
    s = lax.axis_index("s")
    wid = s * NC + c
    per_tile = ex.shape[0] // 16 // NW
    n_chunks = per_tile // B
    zeros16 = jnp.zeros((16,), jnp.float32)
    lanes = lax.broadcasted_iota(jnp.int32, (16,), 0)

    pltpu.sync_copy(attf, attv)

    @pl.loop(0, F)
    def _(k):
        attb[pl.ds(k * 16, 16)] = plsc.load_gather(
            attv, [jnp.full((16,), k, jnp.int32)])

    @pl.loop(0, B)
    def _(i):
        exb[pl.ds(i * 16, 16)] = zeros16

    @pl.loop(0, RPT)
    def _(i):
        zb[pl.ds(i * 16, 16)] = zeros16

    pltpu.sync_copy(zb, den_sh.at[pl.ds(s * RPT * 16, RPT * 16)])
    plsc.subcore_barrier()

    @pl.loop(0, n_chunks)
    def _(k):
        base = wid * per_tile + k * B
        pltpu.sync_copy(src.at[pl.ds(base, B)], srcb)
        pltpu.sync_copy(dstg.at[pl.ds(base, B)], dgb)
        pltpu.sync_copy(dsts.at[pl.ds(base, B)], dsb)
        cp0 = pltpu.async_copy(xl.at[srcb], xlb, sem0)
        cp1 = pltpu.async_copy(xr.at[dgb], xrb, sem1)

        @pl.loop(0, B)
        def _(e):
            dv = plsc.load_gather(dsb, [jnp.full((16,), e, jnp.int32)])
            idxb[pl.ds(e * 16, 16)] = dv * 16 + lanes

        cp0.wait()
        cp1.wait()

        @pl.loop(0, B // 16)
        def _(g):
            eidx = g * 16 + lanes

            @pl.loop(0, HEADS)
            def _(h):
                acc = zeros16
                for d in range(HID):
                    col = h * HID + d
                    cf = jnp.full((16,), col, jnp.int32)
                    a = plsc.load_gather(xlb, [eidx, cf])
                    bv = plsc.load_gather(xrb, [eidx, cf])
                    m = a + bv
                    m = jnp.where(m > 0, m, NEG * m)
                    acc = acc + m * attb[pl.ds(col * 16, 16)]
                exv = jnp.exp(acc)
                plsc.store_scatter(exb, [eidx * 16 + h], exv)

        pltpu.sync_copy(exb, ex.at[pl.ds(base * 16, B * 16)])
        pltpu.sync_copy(exb, den_sh.at[idxb], add=True)

    plsc.subcore_barrier()
    r0 = s * RPT * 16
    pltpu.sync_copy(den_sh.at[pl.ds(r0, RPT * 16)], zb)
    pltpu.sync_copy(zb, den.at[c, pl.ds(r0, RPT * 16)])


def _agg_body(xl, src, dsts, ex,
              out,
              xlb, ob, exb, srcb, dsb, out_sh,
              sem0):
    c = lax.axis_index("c")
    s = lax.axis_index("s")
    wid = s * NC + c
    per_tile = ex.shape[0] // 16 // NW
    n_chunks = per_tile // B
    zeros16 = jnp.zeros((16,), jnp.float32)

    @pl.loop(0, B)
    def _(i):
        for j in range(8):
            ob[i, pl.ds(j * 16, 16)] = zeros16

    r0 = s * RPT
    for t in range(RPT // B):
        pltpu.sync_copy(ob, out_sh.at[pl.ds(r0 + t * B, B)])
    plsc.subcore_barrier()

    @pl.loop(0, n_chunks)
    def _(k):
        base = wid * per_tile + k * B
        pltpu.sync_copy(src.at[pl.ds(base, B)], srcb)
        pltpu.sync_copy(dsts.at[pl.ds(base, B)], dsb)
        cp0 = pltpu.async_copy(xl.at[srcb], xlb, sem0)
        pltpu.sync_copy(ex.at[pl.ds(base * 16, B * 16)], exb)
        cp0.wait()

        @pl.loop(0, B)
        def _(e):
            alv = exb[pl.ds(e * 16, 16)]
            for h in range(HEADS):
                ah = jnp.full((16,), alv[h])
                xv = xlb[e, pl.ds(h * 16, 16)]
                ob[e, pl.ds(h * 16, 16)] = ah * xv

        pltpu.sync_copy(ob, out_sh.at[dsb], add=True)

    plsc.subcore_barrier()
    for t in range(RPT // B):
        pltpu.sync_copy(out_sh.at[pl.ds(r0 + t * B, B)], xlb)
        pltpu.sync_copy(xlb, out.at[c, pl.ds(r0 + t * B, B)])


def _make_score(ep):
    return pl.kernel(
        _score_body,
        out_type=[
            jax.ShapeDtypeStruct((ep * 16,), jnp.float32),
            jax.ShapeDtypeStruct((NC, NP * 16), jnp.float32),
        ],
        mesh=_mesh,
        compiler_params=_SC_PARAMS,
        scratch_types=[
            pltpu.VMEM((B, F), jnp.float32),
            pltpu.VMEM((B, F), jnp.float32),
            pltpu.VMEM((B * 16,), jnp.float32),
            pltpu.VMEM((B,), jnp.int32),
            pltpu.VMEM((B,), jnp.int32),
            pltpu.VMEM((B,), jnp.int32),
            pltpu.VMEM((B * 16,), jnp.int32),
            pltpu.VMEM((F,), jnp.float32),
            pltpu.VMEM((F * 16,), jnp.float32),
            pltpu.VMEM((RPT * 16,), jnp.float32),
            pltpu.VMEM_SHARED((NP * 16,), jnp.float32),
            pltpu.SemaphoreType.DMA,
            pltpu.SemaphoreType.DMA,
        ],
    )


def _make_agg(ep):
    return pl.kernel(
        _agg_body,
        out_type=jax.ShapeDtypeStruct((NC, NP, F), jnp.float32),
        mesh=_mesh,
        compiler_params=_SC_PARAMS,
        scratch_types=[
            pltpu.VMEM((B, F), jnp.float32),
            pltpu.VMEM((B, F), jnp.float32),
            pltpu.VMEM((B * 16,), jnp.float32),
            pltpu.VMEM((B,), jnp.int32),
            pltpu.VMEM((B,), jnp.int32),
            pltpu.VMEM_SHARED((NP, F), jnp.float32),
            pltpu.SemaphoreType.DMA,
        ],
    )


def _mm2_body(x_ref, wl_ref, wr_ref, xl_ref, xr_ref):
    x = x_ref[...]
    xl_ref[...] = jnp.dot(x, wl_ref[...], preferred_element_type=jnp.float32)
    xr_ref[...] = jnp.dot(x, wr_ref[...], preferred_element_type=jnp.float32)


def _mid_body(o_ref, d_ref, r_ref, b0_ref, g0_ref, bb0_ref,
              wl1_ref, wr1_ref, xl1_ref, xr1_ref):
    raw = o_ref[0, pl.ds(0, N), :] + o_ref[1, pl.ds(0, N), :]
    den = d_ref[0, pl.ds(0, N), :] + d_ref[1, pl.ds(0, N), :]
    dexp = jnp.dot(den, r_ref[...], preferred_element_type=jnp.float32)
    h = raw / (dexp + 1e-16) + b0_ref[...]
    mu = jnp.mean(h, axis=0)
    xc = h - mu
    var = jnp.mean(xc * xc, axis=0)
    hn = xc * lax.rsqrt(var + EPS) * g0_ref[...] + bb0_ref[...]
    he = jnp.where(hn > 0, hn, jnp.exp(hn) - 1.0)
    xl1_ref[...] = jnp.dot(he, wl1_ref[...],
                           preferred_element_type=jnp.float32)
    xr1_ref[...] = jnp.dot(he, wr1_ref[...],
                           preferred_element_type=jnp.float32)


def _fin_body(o_ref, d_ref, r_ref, m_ref, b1_ref, g1_ref,
              bb1_ref, cw_ref, cb_ref, out_ref):
    raw = o_ref[0, pl.ds(0, N), :] + o_ref[1, pl.ds(0, N), :]
    den = d_ref[0, pl.ds(0, N), :] + d_ref[1, pl.ds(0, N), :]
    dexp = jnp.dot(den, r_ref[...], preferred_element_type=jnp.float32)
    hm = raw / (dexp + 1e-16)
    hv = jnp.dot(hm, m_ref[...], preferred_element_type=jnp.float32)
    hv = hv + b1_ref[...]
    mu = jnp.mean(hv, axis=0)
    xc = hv - mu
    var = jnp.mean(xc * xc, axis=0)
    hn = xc * lax.rsqrt(var + EPS) * g1_ref[...] + bb1_ref[...]
    out_ref[...] = jnp.dot(hn, cw_ref[...],
                           preferred_element_type=jnp.float32) + cb_ref[...]


@jax.jit
def kernel(x, edge_index, conv0_Wl, conv0_Wr, conv0_att, conv0_b, bn0_g,
           bn0_b, conv1_Wl, conv1_Wr, conv1_att, conv1_b, bn1_g, bn1_b,
           cls_W, cls_b):
    e = edge_index.shape[1]
    et = e + N
    ep = ((et + NW * B - 1) // (NW * B)) * (NW * B)
    pad = ep - et

    ei = edge_index.astype(jnp.int32)
    loops = jnp.arange(N, dtype=jnp.int32)
    zpad = jnp.zeros((pad,), jnp.int32)
    src = jnp.concatenate([ei[0], loops, zpad])
    dstg = jnp.concatenate([ei[1], loops, zpad])
    dsts = jnp.concatenate([ei[1], loops, jnp.full((pad,), N, jnp.int32)])

    mm2 = pl.pallas_call(
        _mm2_body,
        out_shape=[jax.ShapeDtypeStruct((N, F), jnp.float32)] * 2,
    )
    score = _make_score(ep)
    agg = _make_agg(ep)

    rmat = jnp.zeros((16, F), jnp.float32)
    rmat = rmat.at[jnp.repeat(jnp.arange(8), 16),
                   jnp.arange(F)].set(1.0)
    mmat = jnp.tile(jnp.eye(HID, dtype=jnp.float32), (HEADS, 1)) / HEADS

    xl0, xr0 = mm2(x, conv0_Wl, conv0_Wr)
    ex0, den0 = score(xl0, xr0, src, dstg, dsts, conv0_att.reshape(-1))
    o0 = agg(xl0, src, dsts, ex0)

    mid = pl.pallas_call(
        _mid_body,
        out_shape=[jax.ShapeDtypeStruct((N, F), jnp.float32)] * 2,
    )
    xl1, xr1 = mid(o0, den0.reshape(NC, NP, 16),
                   rmat, conv0_b, bn0_g, bn0_b, conv1_Wl, conv1_Wr)

    ex1, den1 = score(xl1, xr1, src, dstg, dsts, conv1_att.reshape(-1))
    o1 = agg(xl1, src, dsts, ex1)

    fin = pl.pallas_call(
        _fin_body,
        out_shape=jax.ShapeDtypeStruct((N, 2), jnp.float32),
    )
    return fin(o1, den1.reshape(NC, NP, 16),
               rmat, mmat, conv1_b, bn1_g, bn1_b, cls_W, cls_b)
